# baseline (speedup 1.0000x reference)
import functools
import jax
import jax.numpy as jnp
from jax import lax
from jax.experimental import pallas as pl
from jax.experimental.pallas import tpu as pltpu
from jax.experimental.pallas import tpu_sc as plsc

N = 10000
E = 320000
DF = 128
DE = 16
H = 128
G = 64

NC = 2
NS = 16
NW = NC * NS
EW = E // NW
CH = 80
NCHUNK = EW // CH

f32 = jnp.float32


def _sc_mesh():
    return plsc.VectorSubcoreMesh(
        core_axis_name="c", subcore_axis_name="s", num_cores=NC, num_subcores=NS
    )


NBUF = 8
SNBUF = 2


def _sc_gather(table, idx3):
    @functools.partial(
        pl.kernel,
        out_type=jax.ShapeDtypeStruct((E, H), f32),
        mesh=_sc_mesh(),
        scratch_types=[pltpu.VMEM((NCHUNK, CH), jnp.int32)]
        + [pltpu.VMEM((CH, H), f32)] * NBUF
        + [pltpu.SemaphoreType.DMA] * (2 * NBUF),
    )
    def k(table_hbm, idx_hbm, out_hbm, idxv, *rest):
        bufs = rest[:NBUF]
        gsem = rest[NBUF : 2 * NBUF]
        ssem = rest[2 * NBUF :]
        cid = lax.axis_index("c")
        sid = lax.axis_index("s")
        w = cid * NS + sid
        base = w * EW
        pltpu.sync_copy(idx_hbm.at[w], idxv)

        def wait_g(b):
            pltpu.make_async_copy(table_hbm.at[idxv.at[0]], bufs[b], gsem[b]).wait()

        def wait_s(b):
            pltpu.make_async_copy(bufs[b], out_hbm.at[pl.ds(0, CH)], ssem[b]).wait()

        for b in range(NBUF):
            pltpu.async_copy(table_hbm.at[idxv.at[b]], bufs[b], gsem[b])

        def body(k2, _):
            for b in range(NBUF):
                j = NBUF * k2 + b

                @pl.when(j < NCHUNK)
                def _store():
                    wait_g(b)
                    pltpu.async_copy(
                        bufs[b], out_hbm.at[pl.ds(base + j * CH, CH)], ssem[b]
                    )

                @pl.when(j + NBUF < NCHUNK)
                def _next():
                    wait_s(b)
                    pltpu.async_copy(
                        table_hbm.at[idxv.at[j + NBUF]], bufs[b], gsem[b]
                    )

            return _

        lax.fori_loop(0, (NCHUNK + NBUF - 1) // NBUF, body, None)
        for b in range(NBUF):
            wait_s(b)

    return k(table, idx3)


def _sc_segsum(h, dst3):
    @functools.partial(
        pl.kernel,
        out_type=jax.ShapeDtypeStruct((NC, N, H), f32),
        mesh=_sc_mesh(),
        scratch_types=[
            pltpu.VMEM_SHARED((N, H), f32),
            pltpu.VMEM((NCHUNK, CH), jnp.int32),
            pltpu.VMEM((CH, H), f32),
        ]
        + [pltpu.VMEM((CH, H), f32)] * SNBUF
        + [pltpu.SemaphoreType.DMA] * (2 * SNBUF),
    )
    def k(h_hbm, dst_hbm, out_hbm, macc, dstv, zbuf, *rest):
        bufs = rest[:SNBUF]
        lsem = rest[SNBUF : 2 * SNBUF]
        asem = rest[2 * SNBUF :]
        cid = lax.axis_index("c")
        sid = lax.axis_index("s")
        w = cid * NS + sid

        zero16 = jnp.zeros((16,), f32)

        def zfill(r, _):
            for u in range(H // 16):
                zbuf[r, pl.ds(u * 16, 16)] = zero16
            return _

        lax.fori_loop(0, CH, zfill, None)

        nz = N // CH

        def zchunk(kk, _):
            c = sid + kk * NS

            @pl.when(c < nz)
            def _():
                pltpu.sync_copy(zbuf, macc.at[pl.ds(c * CH, CH)])

            return _

        lax.fori_loop(0, (nz + NS - 1) // NS, zchunk, None)
        plsc.subcore_barrier()

        pltpu.sync_copy(dst_hbm.at[w], dstv)
        base = w * EW

        def wait_l(b):
            pltpu.make_async_copy(h_hbm.at[pl.ds(0, CH)], bufs[b], lsem[b]).wait()

        def wait_a(b):
            pltpu.make_async_copy(bufs[b], macc.at[dstv.at[0]], asem[b]).wait()

        for b in range(SNBUF):
            pltpu.async_copy(h_hbm.at[pl.ds(base + b * CH, CH)], bufs[b], lsem[b])

        def body(k2, _):
            for b in range(SNBUF):
                j = SNBUF * k2 + b

                @pl.when(j < NCHUNK)
                def _scat():
                    wait_l(b)
                    pltpu.async_copy(bufs[b], macc.at[dstv.at[j]], asem[b], add=True)

                @pl.when(j + SNBUF < NCHUNK)
                def _next():
                    wait_a(b)
                    pltpu.async_copy(
                        h_hbm.at[pl.ds(base + (j + SNBUF) * CH, CH)], bufs[b], lsem[b]
                    )

            return _

        lax.fori_loop(0, (NCHUNK + SNBUF - 1) // SNBUF, body, None)
        for b in range(SNBUF):
            wait_a(b)
        plsc.subcore_barrier()

        def wchunk(kk, _):
            c = sid + kk * NS

            @pl.when(c < nz)
            def _():
                pltpu.sync_copy(macc.at[pl.ds(c * CH, CH)], bufs[0])
                pltpu.sync_copy(bufs[0], out_hbm.at[cid, pl.ds(c * CH, CH)])

            return _

        lax.fori_loop(0, (nz + NS - 1) // NS, wchunk, None)

    return k(h, dst3)


def _mm_body(a_ref, w_ref, o_ref):
    o_ref[...] = jnp.dot(a_ref[...], w_ref[...], preferred_element_type=f32)


def _tc_mm(a, wT, tile):
    M = a.shape[0]
    K = a.shape[1]
    return pl.pallas_call(
        _mm_body,
        grid=(M // tile,),
        in_specs=[
            pl.BlockSpec((tile, K), lambda i: (i, 0)),
            pl.BlockSpec((K, H), lambda i: (0, 0)),
        ],
        out_specs=pl.BlockSpec((tile, H), lambda i: (i, 0)),
        out_shape=jax.ShapeDtypeStruct((M, H), f32),
    )(a, wT)


def _edge_init_body(gx_ref, ea_ref, w_ref, o_ref):
    o_ref[...] = jnp.maximum(
        gx_ref[...] + jnp.dot(ea_ref[...], w_ref[...], preferred_element_type=f32),
        0.0,
    )


def _tc_edge_init(gx, ea, w1eT, tile):
    return pl.pallas_call(
        _edge_init_body,
        grid=(E // tile,),
        in_specs=[
            pl.BlockSpec((tile, H), lambda i: (i, 0)),
            pl.BlockSpec((tile, DE), lambda i: (i, 0)),
            pl.BlockSpec((DE, H), lambda i: (0, 0)),
        ],
        out_specs=pl.BlockSpec((tile, H), lambda i: (i, 0)),
        out_shape=jax.ShapeDtypeStruct((E, H), f32),
    )(gx, ea, w1eT)


def _mm_partial_body(p_ref, w_ref, o_ref):
    m = p_ref[0] + p_ref[1]
    o_ref[...] = jnp.dot(m, w_ref[...], preferred_element_type=f32)


def _tc_mm_partials(p, wT, tile):
    return pl.pallas_call(
        _mm_partial_body,
        grid=(N // tile,),
        in_specs=[
            pl.BlockSpec((NC, tile, H), lambda i: (0, i, 0)),
            pl.BlockSpec((H, H), lambda i: (0, 0)),
        ],
        out_specs=pl.BlockSpec((tile, H), lambda i: (i, 0)),
        out_shape=jax.ShapeDtypeStruct((N, H), f32),
    )(p, wT)


def _combine_body(h0_ref, gs_ref, gr_ref, w_ref, o_ref):
    mdirW = gs_ref[...] - jnp.dot(gr_ref[...], w_ref[...], preferred_element_type=f32)
    o_ref[...] = jnp.maximum(h0_ref[...] + mdirW, 0.0)


def _tc_combine(h0, gs, gr, w2T, tile):
    return pl.pallas_call(
        _combine_body,
        grid=(E // tile,),
        in_specs=[
            pl.BlockSpec((tile, H), lambda i: (i, 0)),
            pl.BlockSpec((tile, H), lambda i: (i, 0)),
            pl.BlockSpec((tile, H), lambda i: (i, 0)),
            pl.BlockSpec((H, H), lambda i: (0, 0)),
        ],
        out_specs=pl.BlockSpec((tile, H), lambda i: (i, 0)),
        out_shape=jax.ShapeDtypeStruct((E, H), f32),
    )(h0, gs, gr, w2T)


def _final_body(x_ref, p_ref, b_ref, w3x_ref, w3v_ref, b3_ref, o_ref, sums, counts):
    i = pl.program_id(0)
    nblk = pl.num_programs(0)
    tile = x_ref.shape[0]

    @pl.when(i == 0)
    def _():
        sums[...] = jnp.zeros_like(sums)
        counts[...] = jnp.zeros_like(counts)

    v = p_ref[0] + p_ref[1]
    z = (
        jnp.dot(x_ref[...], w3x_ref[...], preferred_element_type=f32)
        + jnp.dot(v, w3v_ref[...], preferred_element_type=f32)
        + b3_ref[...]
    )
    z = jnp.maximum(z, 0.0)
    onehot = (
        b_ref[...] == lax.broadcasted_iota(jnp.int32, (tile, G), 1)
    ).astype(f32)
    sums[...] += lax.dot_general(
        onehot, z, (((0,), (0,)), ((), ())), preferred_element_type=f32
    )
    counts[...] += lax.dot_general(
        onehot, jnp.ones((tile, H), f32), (((0,), (0,)), ((), ())),
        preferred_element_type=f32,
    )

    @pl.when(i == nblk - 1)
    def _():
        o_ref[...] = sums[...] / jnp.maximum(counts[...], 1.0)


def _tc_final(x, p, batch2, w3xT, w3vT, b3r, tile):
    return pl.pallas_call(
        _final_body,
        grid=(N // tile,),
        in_specs=[
            pl.BlockSpec((tile, DF), lambda i: (i, 0)),
            pl.BlockSpec((NC, tile, H), lambda i: (0, i, 0)),
            pl.BlockSpec((tile, 1), lambda i: (i, 0)),
            pl.BlockSpec((DF, H), lambda i: (0, 0)),
            pl.BlockSpec((H, H), lambda i: (0, 0)),
            pl.BlockSpec((1, H), lambda i: (0, 0)),
        ],
        out_specs=pl.BlockSpec((G, H), lambda i: (0, 0)),
        out_shape=jax.ShapeDtypeStruct((G, H), f32),
        scratch_shapes=[pltpu.VMEM((G, H), f32), pltpu.VMEM((G, H), f32)],
    )(x, p, batch2, w3xT, w3vT, b3r)


def kernel(x, edge_index, revedge_index, edge_attr, num_nodes, batch, W1, W2, W3, b3):
    src = edge_index[0]
    dst = edge_index[1] + (jnp.asarray(num_nodes, jnp.int32) - N)

    src3 = src.reshape(NW, NCHUNK, CH)
    dst3 = dst.reshape(NW, NCHUNK, CH)
    rev3 = revedge_index.reshape(NW, NCHUNK, CH)

    w1xT = W1[:, :DF].T
    w1eT = W1[:, DF:].T
    w2T = W2.T
    w3xT = W3[:, :DF].T
    w3vT = W3[:, DF:].T
    b3r = b3.reshape(1, H)
    batch2 = batch.reshape(N, 1)

    TE = 2000
    TN = 2000

    xW1 = _tc_mm(x, w1xT, TN)
    gx = _sc_gather(xW1, src3)
    h0 = _tc_edge_init(gx, edge_attr, w1eT, TE)

    h = h0
    for _ in range(2):
        p = _sc_segsum(h, dst3)
        mW2 = _tc_mm_partials(p, w2T, TN)
        gs = _sc_gather(mW2, src3)
        gr = _sc_gather(h, rev3)
        h = _tc_combine(h0, gs, gr, w2T, TE)

    p = _sc_segsum(h, dst3)
    return _tc_final(x, p, batch2, w3xT, w3vT, b3r, 400)


# speedup vs baseline: 1.0383x; 1.0383x over previous
import functools
import jax
import jax.numpy as jnp
from jax import lax
from jax.experimental import pallas as pl
from jax.experimental.pallas import tpu as pltpu
from jax.experimental.pallas import tpu_sc as plsc

N = 10000
E = 320000
DF = 128
DE = 16
H = 128
G = 64

NC = 2
NS = 16
NW = NC * NS
EW = E // NW
CH = 80
NCHUNK = EW // CH

f32 = jnp.float32


def _sc_mesh():
    return plsc.VectorSubcoreMesh(
        core_axis_name="c", subcore_axis_name="s", num_cores=NC, num_subcores=NS
    )


NBUF = 4
SNBUF = 3


def _sc_gather(table, idx3):
    @functools.partial(
        pl.kernel,
        out_type=jax.ShapeDtypeStruct((E, H), f32),
        mesh=_sc_mesh(),
        scratch_types=[pltpu.VMEM((NCHUNK, CH), jnp.int32)]
        + [pltpu.VMEM((CH, H), f32)] * NBUF
        + [pltpu.SemaphoreType.DMA] * (2 * NBUF),
    )
    def k(table_hbm, idx_hbm, out_hbm, idxv, *rest):
        bufs = rest[:NBUF]
        gsem = rest[NBUF : 2 * NBUF]
        ssem = rest[2 * NBUF :]
        cid = lax.axis_index("c")
        sid = lax.axis_index("s")
        w = cid * NS + sid
        base = w * EW
        pltpu.sync_copy(idx_hbm.at[w], idxv)

        def wait_g(b):
            pltpu.make_async_copy(table_hbm.at[idxv.at[0]], bufs[b], gsem[b]).wait()

        def wait_s(b):
            pltpu.make_async_copy(bufs[b], out_hbm.at[pl.ds(0, CH)], ssem[b]).wait()

        for b in range(NBUF):
            pltpu.async_copy(table_hbm.at[idxv.at[b]], bufs[b], gsem[b])

        def body(k2, _):
            for b in range(NBUF):
                j = NBUF * k2 + b

                @pl.when(j < NCHUNK)
                def _store():
                    wait_g(b)
                    pltpu.async_copy(
                        bufs[b], out_hbm.at[pl.ds(base + j * CH, CH)], ssem[b]
                    )

                @pl.when(j + NBUF < NCHUNK)
                def _next():
                    wait_s(b)
                    pltpu.async_copy(
                        table_hbm.at[idxv.at[j + NBUF]], bufs[b], gsem[b]
                    )

            return _

        lax.fori_loop(0, (NCHUNK + NBUF - 1) // NBUF, body, None)
        for b in range(NBUF):
            wait_s(b)

    return k(table, idx3)


def _sc_segsum(h, dst3):
    @functools.partial(
        pl.kernel,
        out_type=jax.ShapeDtypeStruct((NC, N, H), f32),
        mesh=_sc_mesh(),
        scratch_types=[
            pltpu.VMEM_SHARED((N, H), f32),
            pltpu.VMEM((NCHUNK, CH), jnp.int32),
        ]
        + [pltpu.VMEM((CH, H), f32)] * SNBUF
        + [pltpu.SemaphoreType.DMA] * (2 * SNBUF),
    )
    def k(h_hbm, dst_hbm, out_hbm, macc, dstv, *rest):
        bufs = rest[:SNBUF]
        lsem = rest[SNBUF : 2 * SNBUF]
        asem = rest[2 * SNBUF :]
        cid = lax.axis_index("c")
        sid = lax.axis_index("s")
        w = cid * NS + sid

        base = w * EW
        pltpu.sync_copy(dst_hbm.at[w], dstv)

        zero16 = jnp.zeros((16,), f32)

        def zfill(r, _):
            for u in range(H // 16):
                bufs[0][r, pl.ds(u * 16, 16)] = zero16
            return _

        lax.fori_loop(0, CH, zfill, None)

        nz = N // CH

        def zchunk(kk, _):
            c = sid + kk * NS

            @pl.when(c < nz)
            def _():
                pltpu.sync_copy(bufs[0], macc.at[pl.ds(c * CH, CH)])

            return _

        lax.fori_loop(0, (nz + NS - 1) // NS, zchunk, None)
        plsc.subcore_barrier()

        def wait_l(b):
            pltpu.make_async_copy(h_hbm.at[pl.ds(0, CH)], bufs[b], lsem[b]).wait()

        def wait_a(b):
            pltpu.make_async_copy(bufs[b], macc.at[dstv.at[0]], asem[b]).wait()

        for b in range(SNBUF):
            pltpu.async_copy(h_hbm.at[pl.ds(base + b * CH, CH)], bufs[b], lsem[b])

        def body(k2, _):
            for b in range(SNBUF):
                j = SNBUF * k2 + b

                @pl.when(j < NCHUNK)
                def _scat():
                    wait_l(b)
                    pltpu.async_copy(bufs[b], macc.at[dstv.at[j]], asem[b], add=True)

                @pl.when(j + SNBUF < NCHUNK)
                def _next():
                    wait_a(b)
                    pltpu.async_copy(
                        h_hbm.at[pl.ds(base + (j + SNBUF) * CH, CH)], bufs[b], lsem[b]
                    )

            return _

        lax.fori_loop(0, (NCHUNK + SNBUF - 1) // SNBUF, body, None)
        for b in range(SNBUF):
            wait_a(b)
        plsc.subcore_barrier()

        def wchunk(kk, _):
            c = sid + kk * NS

            @pl.when(c < nz)
            def _():
                pltpu.sync_copy(macc.at[pl.ds(c * CH, CH)], bufs[0])
                pltpu.sync_copy(bufs[0], out_hbm.at[cid, pl.ds(c * CH, CH)])

            return _

        lax.fori_loop(0, (nz + NS - 1) // NS, wchunk, None)

    return k(h, dst3)


def _mm_body(a_ref, w_ref, o_ref):
    o_ref[...] = jnp.dot(a_ref[...], w_ref[...], preferred_element_type=f32)


def _tc_mm(a, wT, tile):
    M = a.shape[0]
    K = a.shape[1]
    return pl.pallas_call(
        _mm_body,
        grid=(M // tile,),
        in_specs=[
            pl.BlockSpec((tile, K), lambda i: (i, 0)),
            pl.BlockSpec((K, H), lambda i: (0, 0)),
        ],
        out_specs=pl.BlockSpec((tile, H), lambda i: (i, 0)),
        out_shape=jax.ShapeDtypeStruct((M, H), f32),
    )(a, wT)


def _edge_init_body(gx_ref, ea_ref, w_ref, o_ref):
    o_ref[...] = jnp.maximum(
        gx_ref[...] + jnp.dot(ea_ref[...], w_ref[...], preferred_element_type=f32),
        0.0,
    )


def _tc_edge_init(gx, ea, w1eT, tile):
    return pl.pallas_call(
        _edge_init_body,
        grid=(E // tile,),
        in_specs=[
            pl.BlockSpec((tile, H), lambda i: (i, 0)),
            pl.BlockSpec((tile, DE), lambda i: (i, 0)),
            pl.BlockSpec((DE, H), lambda i: (0, 0)),
        ],
        out_specs=pl.BlockSpec((tile, H), lambda i: (i, 0)),
        out_shape=jax.ShapeDtypeStruct((E, H), f32),
    )(gx, ea, w1eT)


def _mm_partial_body(p_ref, w_ref, o_ref):
    m = p_ref[0] + p_ref[1]
    o_ref[...] = jnp.dot(m, w_ref[...], preferred_element_type=f32)


def _tc_mm_partials(p, wT, tile):
    return pl.pallas_call(
        _mm_partial_body,
        grid=(N // tile,),
        in_specs=[
            pl.BlockSpec((NC, tile, H), lambda i: (0, i, 0)),
            pl.BlockSpec((H, H), lambda i: (0, 0)),
        ],
        out_specs=pl.BlockSpec((tile, H), lambda i: (i, 0)),
        out_shape=jax.ShapeDtypeStruct((N, H), f32),
    )(p, wT)


def _combine_body(h0_ref, gs_ref, gr_ref, w_ref, o_ref):
    mdirW = gs_ref[...] - jnp.dot(gr_ref[...], w_ref[...], preferred_element_type=f32)
    o_ref[...] = jnp.maximum(h0_ref[...] + mdirW, 0.0)


def _tc_combine(h0, gs, gr, w2T, tile):
    return pl.pallas_call(
        _combine_body,
        grid=(E // tile,),
        in_specs=[
            pl.BlockSpec((tile, H), lambda i: (i, 0)),
            pl.BlockSpec((tile, H), lambda i: (i, 0)),
            pl.BlockSpec((tile, H), lambda i: (i, 0)),
            pl.BlockSpec((H, H), lambda i: (0, 0)),
        ],
        out_specs=pl.BlockSpec((tile, H), lambda i: (i, 0)),
        out_shape=jax.ShapeDtypeStruct((E, H), f32),
    )(h0, gs, gr, w2T)


def _final_body(x_ref, p_ref, b_ref, w3x_ref, w3v_ref, b3_ref, o_ref, sums, counts):
    i = pl.program_id(0)
    nblk = pl.num_programs(0)
    tile = x_ref.shape[0]

    @pl.when(i == 0)
    def _():
        sums[...] = jnp.zeros_like(sums)
        counts[...] = jnp.zeros_like(counts)

    v = p_ref[0] + p_ref[1]
    z = (
        jnp.dot(x_ref[...], w3x_ref[...], preferred_element_type=f32)
        + jnp.dot(v, w3v_ref[...], preferred_element_type=f32)
        + b3_ref[...]
    )
    z = jnp.maximum(z, 0.0)
    onehot = (
        b_ref[...] == lax.broadcasted_iota(jnp.int32, (tile, G), 1)
    ).astype(f32)
    sums[...] += lax.dot_general(
        onehot, z, (((0,), (0,)), ((), ())), preferred_element_type=f32
    )
    counts[...] += lax.dot_general(
        onehot, jnp.ones((tile, H), f32), (((0,), (0,)), ((), ())),
        preferred_element_type=f32,
    )

    @pl.when(i == nblk - 1)
    def _():
        o_ref[...] = sums[...] / jnp.maximum(counts[...], 1.0)


def _tc_final(x, p, batch2, w3xT, w3vT, b3r, tile):
    return pl.pallas_call(
        _final_body,
        grid=(N // tile,),
        in_specs=[
            pl.BlockSpec((tile, DF), lambda i: (i, 0)),
            pl.BlockSpec((NC, tile, H), lambda i: (0, i, 0)),
            pl.BlockSpec((tile, 1), lambda i: (i, 0)),
            pl.BlockSpec((DF, H), lambda i: (0, 0)),
            pl.BlockSpec((H, H), lambda i: (0, 0)),
            pl.BlockSpec((1, H), lambda i: (0, 0)),
        ],
        out_specs=pl.BlockSpec((G, H), lambda i: (0, 0)),
        out_shape=jax.ShapeDtypeStruct((G, H), f32),
        scratch_shapes=[pltpu.VMEM((G, H), f32), pltpu.VMEM((G, H), f32)],
    )(x, p, batch2, w3xT, w3vT, b3r)


def kernel(x, edge_index, revedge_index, edge_attr, num_nodes, batch, W1, W2, W3, b3):
    src = edge_index[0]
    dst = edge_index[1] + (jnp.asarray(num_nodes, jnp.int32) - N)

    src3 = src.reshape(NW, NCHUNK, CH)
    dst3 = dst.reshape(NW, NCHUNK, CH)
    rev3 = revedge_index.reshape(NW, NCHUNK, CH)

    w1xT = W1[:, :DF].T
    w1eT = W1[:, DF:].T
    w2T = W2.T
    w3xT = W3[:, :DF].T
    w3vT = W3[:, DF:].T
    b3r = b3.reshape(1, H)
    batch2 = batch.reshape(N, 1)

    TE = 2000
    TN = 2000

    xW1 = _tc_mm(x, w1xT, TN)
    gx = _sc_gather(xW1, src3)
    h0 = _tc_edge_init(gx, edge_attr, w1eT, TE)

    h = h0
    for _ in range(2):
        p = _sc_segsum(h, dst3)
        mW2 = _tc_mm_partials(p, w2T, TN)
        gs = _sc_gather(mW2, src3)
        gr = _sc_gather(h, rev3)
        h = _tc_combine(h0, gs, gr, w2T, TE)

    p = _sc_segsum(h, dst3)
    return _tc_final(x, p, batch2, w3xT, w3vT, b3r, 400)


# speedup vs baseline: 1.0433x; 1.0048x over previous
import functools
import jax
import jax.numpy as jnp
from jax import lax
from jax.experimental import pallas as pl
from jax.experimental.pallas import tpu as pltpu
from jax.experimental.pallas import tpu_sc as plsc

N = 10000
E = 320000
DF = 128
DE = 16
H = 128
G = 64

NC = 2
NS = 16
NW = NC * NS
EW = E // NW
CH = 80
NCHUNK = EW // CH

f32 = jnp.float32


def _sc_mesh():
    return plsc.VectorSubcoreMesh(
        core_axis_name="c", subcore_axis_name="s", num_cores=NC, num_subcores=NS
    )


NBUF = 4
SNBUF = 3


def _sc_gather(table, idx3):
    @functools.partial(
        pl.kernel,
        out_type=jax.ShapeDtypeStruct((E, H), f32),
        mesh=_sc_mesh(),
        scratch_types=[pltpu.VMEM((NCHUNK, CH), jnp.int32)]
        + [pltpu.VMEM((CH, H), f32)] * NBUF
        + [pltpu.SemaphoreType.DMA] * (2 * NBUF),
    )
    def k(table_hbm, idx_hbm, out_hbm, idxv, *rest):
        bufs = rest[:NBUF]
        gsem = rest[NBUF : 2 * NBUF]
        ssem = rest[2 * NBUF :]
        cid = lax.axis_index("c")
        sid = lax.axis_index("s")
        w = cid * NS + sid
        base = w * EW
        pltpu.sync_copy(idx_hbm.at[w], idxv)

        def wait_g(b):
            pltpu.make_async_copy(table_hbm.at[idxv.at[0]], bufs[b], gsem[b]).wait()

        def wait_s(b):
            pltpu.make_async_copy(bufs[b], out_hbm.at[pl.ds(0, CH)], ssem[b]).wait()

        for b in range(NBUF):
            pltpu.async_copy(table_hbm.at[idxv.at[b]], bufs[b], gsem[b])

        def body(k2, _):
            for b in range(NBUF):
                j = NBUF * k2 + b

                @pl.when(j < NCHUNK)
                def _store():
                    wait_g(b)
                    pltpu.async_copy(
                        bufs[b], out_hbm.at[pl.ds(base + j * CH, CH)], ssem[b]
                    )

                @pl.when(j + NBUF < NCHUNK)
                def _next():
                    wait_s(b)
                    pltpu.async_copy(
                        table_hbm.at[idxv.at[j + NBUF]], bufs[b], gsem[b]
                    )

            return _

        lax.fori_loop(0, (NCHUNK + NBUF - 1) // NBUF, body, None)
        for b in range(NBUF):
            wait_s(b)

    return k(table, idx3)


def _sc_segsum(h, dst3):
    @functools.partial(
        pl.kernel,
        out_type=jax.ShapeDtypeStruct((NC, N, H), f32),
        mesh=_sc_mesh(),
        scratch_types=[
            pltpu.VMEM_SHARED((N, H), f32),
            pltpu.VMEM((NCHUNK, CH), jnp.int32),
        ]
        + [pltpu.VMEM((CH, H), f32)] * SNBUF
        + [pltpu.SemaphoreType.DMA] * (2 * SNBUF),
    )
    def k(h_hbm, dst_hbm, out_hbm, macc, dstv, *rest):
        bufs = rest[:SNBUF]
        lsem = rest[SNBUF : 2 * SNBUF]
        asem = rest[2 * SNBUF :]
        cid = lax.axis_index("c")
        sid = lax.axis_index("s")
        w = cid * NS + sid

        base = w * EW
        pltpu.sync_copy(dst_hbm.at[w], dstv)

        zero16 = jnp.zeros((16,), f32)

        def zfill(r, _):
            for u in range(H // 16):
                bufs[0][r, pl.ds(u * 16, 16)] = zero16
            return _

        lax.fori_loop(0, CH, zfill, None)

        nz = N // CH

        def zchunk(kk, _):
            c = sid + kk * NS

            @pl.when(c < nz)
            def _():
                pltpu.sync_copy(bufs[0], macc.at[pl.ds(c * CH, CH)])

            return _

        lax.fori_loop(0, (nz + NS - 1) // NS, zchunk, None)
        plsc.subcore_barrier()

        def wait_l(b):
            pltpu.make_async_copy(h_hbm.at[pl.ds(0, CH)], bufs[b], lsem[b]).wait()

        def wait_a(b):
            pltpu.make_async_copy(bufs[b], macc.at[dstv.at[0]], asem[b]).wait()

        for b in range(SNBUF):
            pltpu.async_copy(h_hbm.at[pl.ds(base + b * CH, CH)], bufs[b], lsem[b])

        def body(k2, _):
            for b in range(SNBUF):
                j = SNBUF * k2 + b

                @pl.when(j < NCHUNK)
                def _scat():
                    wait_l(b)
                    pltpu.async_copy(bufs[b], macc.at[dstv.at[j]], asem[b], add=True)

                @pl.when(j + SNBUF < NCHUNK)
                def _next():
                    wait_a(b)
                    pltpu.async_copy(
                        h_hbm.at[pl.ds(base + (j + SNBUF) * CH, CH)], bufs[b], lsem[b]
                    )

            return _

        lax.fori_loop(0, (NCHUNK + SNBUF - 1) // SNBUF, body, None)
        for b in range(SNBUF):
            wait_a(b)
        plsc.subcore_barrier()

        def wchunk(kk, _):
            c = sid + kk * NS

            @pl.when(c < nz)
            def _():
                pltpu.sync_copy(macc.at[pl.ds(c * CH, CH)], bufs[0])
                pltpu.sync_copy(bufs[0], out_hbm.at[cid, pl.ds(c * CH, CH)])

            return _

        lax.fori_loop(0, (nz + NS - 1) // NS, wchunk, None)

    return k(h, dst3)


def _mm_body(a_ref, w_ref, o_ref):
    o_ref[...] = jnp.dot(a_ref[...], w_ref[...], preferred_element_type=f32)


def _tc_mm(a, wT, tile):
    M = a.shape[0]
    K = a.shape[1]
    return pl.pallas_call(
        _mm_body,
        grid=(M // tile,),
        in_specs=[
            pl.BlockSpec((tile, K), lambda i: (i, 0)),
            pl.BlockSpec((K, H), lambda i: (0, 0)),
        ],
        out_specs=pl.BlockSpec((tile, H), lambda i: (i, 0)),
        out_shape=jax.ShapeDtypeStruct((M, H), f32),
    )(a, wT)


def _edge_init_body(gx_ref, ea_ref, w_ref, o_ref, ob_ref):
    z = jnp.maximum(
        gx_ref[...] + jnp.dot(ea_ref[...], w_ref[...], preferred_element_type=f32),
        0.0,
    )
    o_ref[...] = z
    ob_ref[...] = z.astype(jnp.bfloat16)


def _tc_edge_init(gx, ea, w1eT, tile):
    return pl.pallas_call(
        _edge_init_body,
        grid=(E // tile,),
        in_specs=[
            pl.BlockSpec((tile, H), lambda i: (i, 0)),
            pl.BlockSpec((tile, DE), lambda i: (i, 0)),
            pl.BlockSpec((DE, H), lambda i: (0, 0)),
        ],
        out_specs=[
            pl.BlockSpec((tile, H), lambda i: (i, 0)),
            pl.BlockSpec((tile, H), lambda i: (i, 0)),
        ],
        out_shape=[
            jax.ShapeDtypeStruct((E, H), f32),
            jax.ShapeDtypeStruct((E, H), jnp.bfloat16),
        ],
    )(gx, ea, w1eT)


def _mm_partial_body(p_ref, w_ref, o_ref):
    m = p_ref[0] + p_ref[1]
    o_ref[...] = jnp.dot(m, w_ref[...], preferred_element_type=f32)


def _tc_mm_partials(p, wT, tile):
    return pl.pallas_call(
        _mm_partial_body,
        grid=(N // tile,),
        in_specs=[
            pl.BlockSpec((NC, tile, H), lambda i: (0, i, 0)),
            pl.BlockSpec((H, H), lambda i: (0, 0)),
        ],
        out_specs=pl.BlockSpec((tile, H), lambda i: (i, 0)),
        out_shape=jax.ShapeDtypeStruct((N, H), f32),
    )(p, wT)


def _combine_body(h0_ref, gs_ref, gr_ref, w_ref, o_ref):
    mdirW = gs_ref[...] - jnp.dot(gr_ref[...], w_ref[...], preferred_element_type=f32)
    o_ref[...] = jnp.maximum(h0_ref[...].astype(f32) + mdirW, 0.0)


def _tc_combine(h0, gs, gr, w2T, tile):
    return pl.pallas_call(
        _combine_body,
        grid=(E // tile,),
        in_specs=[
            pl.BlockSpec((tile, H), lambda i: (i, 0)),
            pl.BlockSpec((tile, H), lambda i: (i, 0)),
            pl.BlockSpec((tile, H), lambda i: (i, 0)),
            pl.BlockSpec((H, H), lambda i: (0, 0)),
        ],
        out_specs=pl.BlockSpec((tile, H), lambda i: (i, 0)),
        out_shape=jax.ShapeDtypeStruct((E, H), f32),
    )(h0, gs, gr, w2T)


def _final_body(x_ref, p_ref, b_ref, w3x_ref, w3v_ref, b3_ref, o_ref, sums, counts):
    i = pl.program_id(0)
    nblk = pl.num_programs(0)
    tile = x_ref.shape[0]

    @pl.when(i == 0)
    def _():
        sums[...] = jnp.zeros_like(sums)
        counts[...] = jnp.zeros_like(counts)

    v = p_ref[0] + p_ref[1]
    z = (
        jnp.dot(x_ref[...], w3x_ref[...], preferred_element_type=f32)
        + jnp.dot(v, w3v_ref[...], preferred_element_type=f32)
        + b3_ref[...]
    )
    z = jnp.maximum(z, 0.0)
    onehot = (
        b_ref[...] == lax.broadcasted_iota(jnp.int32, (tile, G), 1)
    ).astype(f32)
    sums[...] += lax.dot_general(
        onehot, z, (((0,), (0,)), ((), ())), preferred_element_type=f32
    )
    counts[...] += lax.dot_general(
        onehot, jnp.ones((tile, H), f32), (((0,), (0,)), ((), ())),
        preferred_element_type=f32,
    )

    @pl.when(i == nblk - 1)
    def _():
        o_ref[...] = sums[...] / jnp.maximum(counts[...], 1.0)


def _tc_final(x, p, batch2, w3xT, w3vT, b3r, tile):
    return pl.pallas_call(
        _final_body,
        grid=(N // tile,),
        in_specs=[
            pl.BlockSpec((tile, DF), lambda i: (i, 0)),
            pl.BlockSpec((NC, tile, H), lambda i: (0, i, 0)),
            pl.BlockSpec((tile, 1), lambda i: (i, 0)),
            pl.BlockSpec((DF, H), lambda i: (0, 0)),
            pl.BlockSpec((H, H), lambda i: (0, 0)),
            pl.BlockSpec((1, H), lambda i: (0, 0)),
        ],
        out_specs=pl.BlockSpec((G, H), lambda i: (0, 0)),
        out_shape=jax.ShapeDtypeStruct((G, H), f32),
        scratch_shapes=[pltpu.VMEM((G, H), f32), pltpu.VMEM((G, H), f32)],
    )(x, p, batch2, w3xT, w3vT, b3r)


def kernel(x, edge_index, revedge_index, edge_attr, num_nodes, batch, W1, W2, W3, b3):
    src = edge_index[0]
    dst = edge_index[1] + (jnp.asarray(num_nodes, jnp.int32) - N)

    src3 = src.reshape(NW, NCHUNK, CH)
    dst3 = dst.reshape(NW, NCHUNK, CH)
    rev3 = revedge_index.reshape(NW, NCHUNK, CH)

    w1xT = W1[:, :DF].T
    w1eT = W1[:, DF:].T
    w2T = W2.T
    w3xT = W3[:, :DF].T
    w3vT = W3[:, DF:].T
    b3r = b3.reshape(1, H)
    batch2 = batch.reshape(N, 1)

    TE = 2000
    TN = 2000

    xW1 = _tc_mm(x, w1xT, TN)
    gx = _sc_gather(xW1, src3)
    h0, h0b = _tc_edge_init(gx, edge_attr, w1eT, TE)

    h = h0
    for _ in range(2):
        p = _sc_segsum(h, dst3)
        mW2 = _tc_mm_partials(p, w2T, TN)
        gs = _sc_gather(mW2, src3)
        gr = _sc_gather(h, rev3)
        h = _tc_combine(h0b, gs, gr, w2T, TE)

    p = _sc_segsum(h, dst3)
    return _tc_final(x, p, batch2, w3xT, w3vT, b3r, 400)


# speedup vs baseline: 1.1064x; 1.0605x over previous
import functools
import jax
import jax.numpy as jnp
from jax import lax
from jax.experimental import pallas as pl
from jax.experimental.pallas import tpu as pltpu
from jax.experimental.pallas import tpu_sc as plsc

N = 10000
E = 320000
DF = 128
DE = 16
H = 128
G = 64

NC = 2
NS = 16
NW = NC * NS
EW = E // NW
CH = 80
NCHUNK = EW // CH

f32 = jnp.float32


def _sc_mesh():
    return plsc.VectorSubcoreMesh(
        core_axis_name="c", subcore_axis_name="s", num_cores=NC, num_subcores=NS
    )


NBUF = 4
SNBUF = 3


def _sc_gather(table, idx3):
    @functools.partial(
        pl.kernel,
        out_type=jax.ShapeDtypeStruct((E, H), f32),
        mesh=_sc_mesh(),
        scratch_types=[pltpu.VMEM((NCHUNK, CH), jnp.int32)]
        + [pltpu.VMEM((CH, H), f32)] * NBUF
        + [pltpu.SemaphoreType.DMA] * (2 * NBUF),
    )
    def k(table_hbm, idx_hbm, out_hbm, idxv, *rest):
        bufs = rest[:NBUF]
        gsem = rest[NBUF : 2 * NBUF]
        ssem = rest[2 * NBUF :]
        cid = lax.axis_index("c")
        sid = lax.axis_index("s")
        w = cid * NS + sid
        base = w * EW
        pltpu.sync_copy(idx_hbm.at[w], idxv)

        def wait_g(b):
            pltpu.make_async_copy(table_hbm.at[idxv.at[0]], bufs[b], gsem[b]).wait()

        def wait_s(b):
            pltpu.make_async_copy(bufs[b], out_hbm.at[pl.ds(0, CH)], ssem[b]).wait()

        for b in range(NBUF):
            pltpu.async_copy(table_hbm.at[idxv.at[b]], bufs[b], gsem[b])

        def body(k2, _):
            for b in range(NBUF):
                j = NBUF * k2 + b

                @pl.when(j < NCHUNK)
                def _store():
                    wait_g(b)
                    pltpu.async_copy(
                        bufs[b], out_hbm.at[pl.ds(base + j * CH, CH)], ssem[b]
                    )

                @pl.when(j + NBUF < NCHUNK)
                def _next():
                    wait_s(b)
                    pltpu.async_copy(
                        table_hbm.at[idxv.at[j + NBUF]], bufs[b], gsem[b]
                    )

            return _

        lax.fori_loop(0, (NCHUNK + NBUF - 1) // NBUF, body, None)
        for b in range(NBUF):
            wait_s(b)

    return k(table, idx3)


def _sc_segsum(h, dst3):
    @functools.partial(
        pl.kernel,
        out_type=jax.ShapeDtypeStruct((NC, N, H), f32),
        mesh=_sc_mesh(),
        scratch_types=[
            pltpu.VMEM_SHARED((N, H), f32),
            pltpu.VMEM((NCHUNK, CH), jnp.int32),
        ]
        + [pltpu.VMEM((CH, H), f32)] * SNBUF
        + [pltpu.SemaphoreType.DMA] * (2 * SNBUF),
    )
    def k(h_hbm, dst_hbm, out_hbm, macc, dstv, *rest):
        bufs = rest[:SNBUF]
        lsem = rest[SNBUF : 2 * SNBUF]
        asem = rest[2 * SNBUF :]
        cid = lax.axis_index("c")
        sid = lax.axis_index("s")
        w = cid * NS + sid

        base = w * EW
        pltpu.sync_copy(dst_hbm.at[w], dstv)

        zero16 = jnp.zeros((16,), f32)

        def zfill(r, _):
            for u in range(H // 16):
                bufs[0][r, pl.ds(u * 16, 16)] = zero16
            return _

        lax.fori_loop(0, CH, zfill, None)

        nz = N // CH

        def zchunk(kk, _):
            c = sid + kk * NS

            @pl.when(c < nz)
            def _():
                pltpu.sync_copy(bufs[0], macc.at[pl.ds(c * CH, CH)])

            return _

        lax.fori_loop(0, (nz + NS - 1) // NS, zchunk, None)
        plsc.subcore_barrier()

        def wait_l(b):
            pltpu.make_async_copy(h_hbm.at[pl.ds(0, CH)], bufs[b], lsem[b]).wait()

        def wait_a(b):
            pltpu.make_async_copy(bufs[b], macc.at[dstv.at[0]], asem[b]).wait()

        for b in range(SNBUF):
            pltpu.async_copy(h_hbm.at[pl.ds(base + b * CH, CH)], bufs[b], lsem[b])

        def body(k2, _):
            for b in range(SNBUF):
                j = SNBUF * k2 + b

                @pl.when(j < NCHUNK)
                def _scat():
                    wait_l(b)
                    pltpu.async_copy(bufs[b], macc.at[dstv.at[j]], asem[b], add=True)

                @pl.when(j + SNBUF < NCHUNK)
                def _next():
                    wait_a(b)
                    pltpu.async_copy(
                        h_hbm.at[pl.ds(base + (j + SNBUF) * CH, CH)], bufs[b], lsem[b]
                    )

            return _

        lax.fori_loop(0, (NCHUNK + SNBUF - 1) // SNBUF, body, None)
        for b in range(SNBUF):
            wait_a(b)
        plsc.subcore_barrier()

        def wchunk(kk, _):
            c = sid + kk * NS

            @pl.when(c < nz)
            def _():
                pltpu.sync_copy(macc.at[pl.ds(c * CH, CH)], bufs[0])
                pltpu.sync_copy(bufs[0], out_hbm.at[cid, pl.ds(c * CH, CH)])

            return _

        lax.fori_loop(0, (nz + NS - 1) // NS, wchunk, None)

    return k(h, dst3)


def _mm_body(a_ref, w_ref, o_ref):
    o_ref[...] = jnp.dot(a_ref[...], w_ref[...], preferred_element_type=f32)


def _tc_mm(a, wT, tile):
    M = a.shape[0]
    K = a.shape[1]
    return pl.pallas_call(
        _mm_body,
        grid=(M // tile,),
        in_specs=[
            pl.BlockSpec((tile, K), lambda i: (i, 0)),
            pl.BlockSpec((K, H), lambda i: (0, 0)),
        ],
        out_specs=pl.BlockSpec((tile, H), lambda i: (i, 0)),
        out_shape=jax.ShapeDtypeStruct((M, H), f32),
    )(a, wT)


def _edge_init_body(gx_ref, ea_ref, w_ref, o_ref, ob_ref):
    z = jnp.maximum(
        gx_ref[...] + jnp.dot(ea_ref[...], w_ref[...], preferred_element_type=f32),
        0.0,
    )
    o_ref[...] = z
    ob_ref[...] = z.astype(jnp.bfloat16)


def _tc_edge_init(gx, ea, w1eT, tile):
    return pl.pallas_call(
        _edge_init_body,
        grid=(E // tile,),
        in_specs=[
            pl.BlockSpec((tile, H), lambda i: (i, 0)),
            pl.BlockSpec((tile, DE), lambda i: (i, 0)),
            pl.BlockSpec((DE, H), lambda i: (0, 0)),
        ],
        out_specs=[
            pl.BlockSpec((tile, H), lambda i: (i, 0)),
            pl.BlockSpec((tile, H), lambda i: (i, 0)),
        ],
        out_shape=[
            jax.ShapeDtypeStruct((E, H), f32),
            jax.ShapeDtypeStruct((E, H), jnp.bfloat16),
        ],
    )(gx, ea, w1eT)


def _mm_partial_body(p_ref, w_ref, o_ref):
    m = p_ref[0] + p_ref[1]
    o_ref[...] = jnp.dot(m, w_ref[...], preferred_element_type=f32)


def _tc_mm_partials(p, wT, tile):
    return pl.pallas_call(
        _mm_partial_body,
        grid=(N // tile,),
        in_specs=[
            pl.BlockSpec((NC, tile, H), lambda i: (0, i, 0)),
            pl.BlockSpec((H, H), lambda i: (0, 0)),
        ],
        out_specs=pl.BlockSpec((tile, H), lambda i: (i, 0)),
        out_shape=jax.ShapeDtypeStruct((N, H), f32),
    )(p, wT)


def _combine_body(h0_ref, gs_ref, gr_ref, w_ref, o_ref):
    mdirW = gs_ref[...] - jnp.dot(gr_ref[...], w_ref[...], preferred_element_type=f32)
    o_ref[...] = jnp.maximum(h0_ref[...].astype(f32) + mdirW, 0.0)


def _tc_combine(h0, gs, gr, w2T, tile):
    return pl.pallas_call(
        _combine_body,
        grid=(E // tile,),
        in_specs=[
            pl.BlockSpec((tile, H), lambda i: (i, 0)),
            pl.BlockSpec((tile, H), lambda i: (i, 0)),
            pl.BlockSpec((tile, H), lambda i: (i, 0)),
            pl.BlockSpec((H, H), lambda i: (0, 0)),
        ],
        out_specs=pl.BlockSpec((tile, H), lambda i: (i, 0)),
        out_shape=jax.ShapeDtypeStruct((E, H), f32),
    )(h0, gs, gr, w2T)


def _final_body(x_ref, p_ref, b_ref, w3x_ref, w3v_ref, b3_ref, o_ref, sums, counts):
    i = pl.program_id(0)
    nblk = pl.num_programs(0)
    tile = x_ref.shape[0]

    @pl.when(i == 0)
    def _():
        sums[...] = jnp.zeros_like(sums)
        counts[...] = jnp.zeros_like(counts)

    v = p_ref[0] + p_ref[1]
    z = (
        jnp.dot(x_ref[...], w3x_ref[...], preferred_element_type=f32)
        + jnp.dot(v, w3v_ref[...], preferred_element_type=f32)
        + b3_ref[...]
    )
    z = jnp.maximum(z, 0.0)
    onehot = (
        b_ref[...] == lax.broadcasted_iota(jnp.int32, (tile, G), 1)
    ).astype(f32)
    sums[...] += lax.dot_general(
        onehot, z, (((0,), (0,)), ((), ())), preferred_element_type=f32
    )
    counts[...] += lax.dot_general(
        onehot, jnp.ones((tile, H), f32), (((0,), (0,)), ((), ())),
        preferred_element_type=f32,
    )

    @pl.when(i == nblk - 1)
    def _():
        o_ref[...] = sums[...] / jnp.maximum(counts[...], 1.0)


def _tc_final(x, p, batch2, w3xT, w3vT, b3r, tile):
    return pl.pallas_call(
        _final_body,
        grid=(N // tile,),
        in_specs=[
            pl.BlockSpec((tile, DF), lambda i: (i, 0)),
            pl.BlockSpec((NC, tile, H), lambda i: (0, i, 0)),
            pl.BlockSpec((tile, 1), lambda i: (i, 0)),
            pl.BlockSpec((DF, H), lambda i: (0, 0)),
            pl.BlockSpec((H, H), lambda i: (0, 0)),
            pl.BlockSpec((1, H), lambda i: (0, 0)),
        ],
        out_specs=pl.BlockSpec((G, H), lambda i: (0, 0)),
        out_shape=jax.ShapeDtypeStruct((G, H), f32),
        scratch_shapes=[pltpu.VMEM((G, H), f32), pltpu.VMEM((G, H), f32)],
    )(x, p, batch2, w3xT, w3vT, b3r)


def kernel(x, edge_index, revedge_index, edge_attr, num_nodes, batch, W1, W2, W3, b3):
    src = edge_index[0]
    dst = edge_index[1] + (jnp.asarray(num_nodes, jnp.int32) - N)

    src3 = src.reshape(NW, NCHUNK, CH)
    dst3 = dst.reshape(NW, NCHUNK, CH)
    rev3 = revedge_index.reshape(NW, NCHUNK, CH)

    w1xT = W1[:, :DF].T
    w1eT = W1[:, DF:].T
    w2T = W2.T
    w3xT = W3[:, :DF].T
    w3vT = W3[:, DF:].T
    b3r = b3.reshape(1, H)
    batch2 = batch.reshape(N, 1)

    TE = 4000
    TN = 2000

    xW1 = _tc_mm(x, w1xT, TN)
    gx = _sc_gather(xW1, src3)
    h0, h0b = _tc_edge_init(gx, edge_attr, w1eT, TE)

    h = h0
    for _ in range(2):
        p = _sc_segsum(h, dst3)
        mW2 = _tc_mm_partials(p, w2T, TN)
        gs = _sc_gather(mW2, src3)
        gr = _sc_gather(h, rev3)
        h = _tc_combine(h0b, gs, gr, w2T, TE)

    p = _sc_segsum(h, dst3)
    return _tc_final(x, p, batch2, w3xT, w3vT, b3r, 400)


# speedup vs baseline: 1.1097x; 1.0030x over previous
import functools
import jax
import jax.numpy as jnp
from jax import lax
from jax.experimental import pallas as pl
from jax.experimental.pallas import tpu as pltpu
from jax.experimental.pallas import tpu_sc as plsc

N = 10000
E = 320000
DF = 128
DE = 16
H = 128
G = 64

NC = 2
NS = 16
NW = NC * NS
EW = E // NW
CH = 80
NCHUNK = EW // CH

f32 = jnp.float32


def _sc_mesh():
    return plsc.VectorSubcoreMesh(
        core_axis_name="c", subcore_axis_name="s", num_cores=NC, num_subcores=NS
    )


NBUF = 4
SNBUF = 3


def _sc_gather(table, idx3):
    @functools.partial(
        pl.kernel,
        out_type=jax.ShapeDtypeStruct((E, H), f32),
        mesh=_sc_mesh(),
        scratch_types=[pltpu.VMEM((NCHUNK, CH), jnp.int32)]
        + [pltpu.VMEM((CH, H), f32)] * NBUF
        + [pltpu.SemaphoreType.DMA] * (2 * NBUF),
    )
    def k(table_hbm, idx_hbm, out_hbm, idxv, *rest):
        bufs = rest[:NBUF]
        gsem = rest[NBUF : 2 * NBUF]
        ssem = rest[2 * NBUF :]
        cid = lax.axis_index("c")
        sid = lax.axis_index("s")
        w = cid * NS + sid
        base = w * EW
        pltpu.sync_copy(idx_hbm.at[w], idxv)

        def wait_g(b):
            pltpu.make_async_copy(table_hbm.at[idxv.at[0]], bufs[b], gsem[b]).wait()

        def wait_s(b):
            pltpu.make_async_copy(bufs[b], out_hbm.at[pl.ds(0, CH)], ssem[b]).wait()

        for b in range(NBUF):
            pltpu.async_copy(table_hbm.at[idxv.at[b]], bufs[b], gsem[b])

        def body(k2, _):
            for b in range(NBUF):
                j = NBUF * k2 + b

                @pl.when(j < NCHUNK)
                def _store():
                    wait_g(b)
                    pltpu.async_copy(
                        bufs[b], out_hbm.at[pl.ds(base + j * CH, CH)], ssem[b]
                    )

                @pl.when(j + NBUF < NCHUNK)
                def _next():
                    wait_s(b)
                    pltpu.async_copy(
                        table_hbm.at[idxv.at[j + NBUF]], bufs[b], gsem[b]
                    )

            return _

        lax.fori_loop(0, (NCHUNK + NBUF - 1) // NBUF, body, None)
        for b in range(NBUF):
            wait_s(b)

    return k(table, idx3)


def _sc_segsum(h, dst3):
    @functools.partial(
        pl.kernel,
        out_type=jax.ShapeDtypeStruct((NC, N, H), f32),
        mesh=_sc_mesh(),
        scratch_types=[
            pltpu.VMEM_SHARED((N, H), f32),
            pltpu.VMEM((NCHUNK, CH), jnp.int32),
        ]
        + [pltpu.VMEM((CH, H), f32)] * SNBUF
        + [pltpu.SemaphoreType.DMA] * (2 * SNBUF),
    )
    def k(h_hbm, dst_hbm, out_hbm, macc, dstv, *rest):
        bufs = rest[:SNBUF]
        lsem = rest[SNBUF : 2 * SNBUF]
        asem = rest[2 * SNBUF :]
        cid = lax.axis_index("c")
        sid = lax.axis_index("s")
        w = cid * NS + sid

        base = w * EW
        pltpu.sync_copy(dst_hbm.at[w], dstv)

        zero16 = jnp.zeros((16,), f32)

        def zfill(r, _):
            for u in range(H // 16):
                bufs[0][r, pl.ds(u * 16, 16)] = zero16
            return _

        lax.fori_loop(0, CH, zfill, None)

        nz = N // CH

        def zchunk(kk, _):
            c = sid + kk * NS

            @pl.when(c < nz)
            def _():
                pltpu.sync_copy(bufs[0], macc.at[pl.ds(c * CH, CH)])

            return _

        lax.fori_loop(0, (nz + NS - 1) // NS, zchunk, None)
        plsc.subcore_barrier()

        def wait_l(b):
            pltpu.make_async_copy(h_hbm.at[pl.ds(0, CH)], bufs[b], lsem[b]).wait()

        def wait_a(b):
            pltpu.make_async_copy(bufs[b], macc.at[dstv.at[0]], asem[b]).wait()

        for b in range(SNBUF):
            pltpu.async_copy(h_hbm.at[pl.ds(base + b * CH, CH)], bufs[b], lsem[b])

        def body(k2, _):
            for b in range(SNBUF):
                j = SNBUF * k2 + b

                @pl.when(j < NCHUNK)
                def _scat():
                    wait_l(b)
                    pltpu.async_copy(bufs[b], macc.at[dstv.at[j]], asem[b], add=True)

                @pl.when(j + SNBUF < NCHUNK)
                def _next():
                    wait_a(b)
                    pltpu.async_copy(
                        h_hbm.at[pl.ds(base + (j + SNBUF) * CH, CH)], bufs[b], lsem[b]
                    )

            return _

        lax.fori_loop(0, (NCHUNK + SNBUF - 1) // SNBUF, body, None)
        for b in range(SNBUF):
            wait_a(b)
        plsc.subcore_barrier()

        def wchunk(kk, _):
            c = sid + kk * NS

            @pl.when(c < nz)
            def _():
                pltpu.sync_copy(macc.at[pl.ds(c * CH, CH)], bufs[0])
                pltpu.sync_copy(bufs[0], out_hbm.at[cid, pl.ds(c * CH, CH)])

            return _

        lax.fori_loop(0, (nz + NS - 1) // NS, wchunk, None)

    return k(h, dst3)


def _mm_body(a_ref, w_ref, o_ref):
    o_ref[...] = jnp.dot(a_ref[...], w_ref[...], preferred_element_type=f32)


def _tc_mm(a, wT, tile):
    M = a.shape[0]
    K = a.shape[1]
    return pl.pallas_call(
        _mm_body,
        grid=(M // tile,),
        in_specs=[
            pl.BlockSpec((tile, K), lambda i: (i, 0)),
            pl.BlockSpec((K, H), lambda i: (0, 0)),
        ],
        out_specs=pl.BlockSpec((tile, H), lambda i: (i, 0)),
        out_shape=jax.ShapeDtypeStruct((M, H), f32),
    )(a, wT)


def _edge_init_body(gx_ref, ea_ref, w_ref, o_ref, ob_ref):
    z = jnp.maximum(
        gx_ref[...] + jnp.dot(ea_ref[...], w_ref[...], preferred_element_type=f32),
        0.0,
    )
    o_ref[...] = z
    ob_ref[...] = z.astype(jnp.bfloat16)


def _tc_edge_init(gx, ea, w1eT, tile):
    return pl.pallas_call(
        _edge_init_body,
        grid=(E // tile,),
        in_specs=[
            pl.BlockSpec((tile, H), lambda i: (i, 0)),
            pl.BlockSpec((tile, DE), lambda i: (i, 0)),
            pl.BlockSpec((DE, H), lambda i: (0, 0)),
        ],
        out_specs=[
            pl.BlockSpec((tile, H), lambda i: (i, 0)),
            pl.BlockSpec((tile, H), lambda i: (i, 0)),
        ],
        out_shape=[
            jax.ShapeDtypeStruct((E, H), f32),
            jax.ShapeDtypeStruct((E, H), jnp.bfloat16),
        ],
    )(gx, ea, w1eT)


def _mm_partial_body(p_ref, w_ref, o_ref):
    m = p_ref[0] + p_ref[1]
    o_ref[...] = jnp.dot(m, w_ref[...], preferred_element_type=f32)


def _tc_mm_partials(p, wT, tile):
    return pl.pallas_call(
        _mm_partial_body,
        grid=(N // tile,),
        in_specs=[
            pl.BlockSpec((NC, tile, H), lambda i: (0, i, 0)),
            pl.BlockSpec((H, H), lambda i: (0, 0)),
        ],
        out_specs=pl.BlockSpec((tile, H), lambda i: (i, 0)),
        out_shape=jax.ShapeDtypeStruct((N, H), f32),
    )(p, wT)


def _combine_body(h0_ref, gs_ref, gr_ref, w_ref, o_ref):
    mdirW = gs_ref[...] - jnp.dot(gr_ref[...], w_ref[...], preferred_element_type=f32)
    o_ref[...] = jnp.maximum(h0_ref[...].astype(f32) + mdirW, 0.0)


def _tc_combine(h0, gs, gr, w2T, tile):
    return pl.pallas_call(
        _combine_body,
        grid=(E // tile,),
        in_specs=[
            pl.BlockSpec((tile, H), lambda i: (i, 0)),
            pl.BlockSpec((tile, H), lambda i: (i, 0)),
            pl.BlockSpec((tile, H), lambda i: (i, 0)),
            pl.BlockSpec((H, H), lambda i: (0, 0)),
        ],
        out_specs=pl.BlockSpec((tile, H), lambda i: (i, 0)),
        out_shape=jax.ShapeDtypeStruct((E, H), f32),
    )(h0, gs, gr, w2T)


def _final_body(x_ref, p_ref, b_ref, w3x_ref, w3v_ref, b3_ref, o_ref, sums, counts):
    i = pl.program_id(0)
    nblk = pl.num_programs(0)
    tile = x_ref.shape[0]

    @pl.when(i == 0)
    def _():
        sums[...] = jnp.zeros_like(sums)
        counts[...] = jnp.zeros_like(counts)

    v = p_ref[0] + p_ref[1]
    z = (
        jnp.dot(x_ref[...], w3x_ref[...], preferred_element_type=f32)
        + jnp.dot(v, w3v_ref[...], preferred_element_type=f32)
        + b3_ref[...]
    )
    z = jnp.maximum(z, 0.0)
    onehot = (
        b_ref[...] == lax.broadcasted_iota(jnp.int32, (tile, G), 1)
    ).astype(f32)
    sums[...] += lax.dot_general(
        onehot, z, (((0,), (0,)), ((), ())), preferred_element_type=f32
    )
    counts[...] += lax.dot_general(
        onehot, jnp.ones((tile, H), f32), (((0,), (0,)), ((), ())),
        preferred_element_type=f32,
    )

    @pl.when(i == nblk - 1)
    def _():
        o_ref[...] = sums[...] / jnp.maximum(counts[...], 1.0)


def _tc_final(x, p, batch2, w3xT, w3vT, b3r, tile):
    return pl.pallas_call(
        _final_body,
        grid=(N // tile,),
        in_specs=[
            pl.BlockSpec((tile, DF), lambda i: (i, 0)),
            pl.BlockSpec((NC, tile, H), lambda i: (0, i, 0)),
            pl.BlockSpec((tile, 1), lambda i: (i, 0)),
            pl.BlockSpec((DF, H), lambda i: (0, 0)),
            pl.BlockSpec((H, H), lambda i: (0, 0)),
            pl.BlockSpec((1, H), lambda i: (0, 0)),
        ],
        out_specs=pl.BlockSpec((G, H), lambda i: (0, 0)),
        out_shape=jax.ShapeDtypeStruct((G, H), f32),
        scratch_shapes=[pltpu.VMEM((G, H), f32), pltpu.VMEM((G, H), f32)],
    )(x, p, batch2, w3xT, w3vT, b3r)


def kernel(x, edge_index, revedge_index, edge_attr, num_nodes, batch, W1, W2, W3, b3):
    src = edge_index[0]
    dst = edge_index[1] + (jnp.asarray(num_nodes, jnp.int32) - N)

    src3 = src.reshape(NW, NCHUNK, CH)
    dst3 = dst.reshape(NW, NCHUNK, CH)
    rev3 = revedge_index.reshape(NW, NCHUNK, CH)

    w1xT = W1[:, :DF].T
    w1eT = W1[:, DF:].T
    w2T = W2.T
    w3xT = W3[:, :DF].T
    w3vT = W3[:, DF:].T
    b3r = b3.reshape(1, H)
    batch2 = batch.reshape(N, 1)

    TE = 8000
    TN = 2000

    xW1 = _tc_mm(x, w1xT, TN)
    gx = _sc_gather(xW1, src3)
    h0, h0b = _tc_edge_init(gx, edge_attr, w1eT, TE)

    h = h0
    for _ in range(2):
        p = _sc_segsum(h, dst3)
        mW2 = _tc_mm_partials(p, w2T, TN)
        gs = _sc_gather(mW2, src3)
        gr = _sc_gather(h, rev3)
        h = _tc_combine(h0b, gs, gr, w2T, TE)

    p = _sc_segsum(h, dst3)
    return _tc_final(x, p, batch2, w3xT, w3vT, b3r, 400)


# speedup vs baseline: 1.1182x; 1.0077x over previous
import functools
import jax
import jax.numpy as jnp
from jax import lax
from jax.experimental import pallas as pl
from jax.experimental.pallas import tpu as pltpu
from jax.experimental.pallas import tpu_sc as plsc

N = 10000
E = 320000
DF = 128
DE = 16
H = 128
G = 64

NC = 2
NS = 16
NW = NC * NS
EW = E // NW
CH = 80
NCHUNK = EW // CH

f32 = jnp.float32


def _sc_mesh():
    return plsc.VectorSubcoreMesh(
        core_axis_name="c", subcore_axis_name="s", num_cores=NC, num_subcores=NS
    )


NBUF = 4
SNBUF = 3


def _sc_gather(table, idx3):
    @functools.partial(
        pl.kernel,
        out_type=jax.ShapeDtypeStruct((E, H), f32),
        mesh=_sc_mesh(),
        scratch_types=[pltpu.VMEM((NCHUNK, CH), jnp.int32)]
        + [pltpu.VMEM((CH, H), f32)] * NBUF
        + [pltpu.SemaphoreType.DMA] * (2 * NBUF),
    )
    def k(table_hbm, idx_hbm, out_hbm, idxv, *rest):
        bufs = rest[:NBUF]
        gsem = rest[NBUF : 2 * NBUF]
        ssem = rest[2 * NBUF :]
        cid = lax.axis_index("c")
        sid = lax.axis_index("s")
        w = cid * NS + sid
        base = w * EW
        pltpu.sync_copy(idx_hbm.at[w], idxv)

        def wait_g(b):
            pltpu.make_async_copy(table_hbm.at[idxv.at[0]], bufs[b], gsem[b]).wait()

        def wait_s(b):
            pltpu.make_async_copy(bufs[b], out_hbm.at[pl.ds(0, CH)], ssem[b]).wait()

        for b in range(NBUF):
            pltpu.async_copy(table_hbm.at[idxv.at[b]], bufs[b], gsem[b])

        def body(k2, _):
            for b in range(NBUF):
                j = NBUF * k2 + b

                @pl.when(j < NCHUNK)
                def _store():
                    wait_g(b)
                    pltpu.async_copy(
                        bufs[b], out_hbm.at[pl.ds(base + j * CH, CH)], ssem[b]
                    )

                @pl.when(j + NBUF < NCHUNK)
                def _next():
                    wait_s(b)
                    pltpu.async_copy(
                        table_hbm.at[idxv.at[j + NBUF]], bufs[b], gsem[b]
                    )

            return _

        lax.fori_loop(0, (NCHUNK + NBUF - 1) // NBUF, body, None)
        for b in range(NBUF):
            wait_s(b)

    return k(table, idx3)


def _sc_segsum(h, dst3):
    @functools.partial(
        pl.kernel,
        out_type=jax.ShapeDtypeStruct((NC, N, H), f32),
        mesh=_sc_mesh(),
        scratch_types=[
            pltpu.VMEM_SHARED((N, H), f32),
            pltpu.VMEM((NCHUNK, CH), jnp.int32),
        ]
        + [pltpu.VMEM((CH, H), f32)] * SNBUF
        + [pltpu.SemaphoreType.DMA] * (2 * SNBUF),
    )
    def k(h_hbm, dst_hbm, out_hbm, macc, dstv, *rest):
        bufs = rest[:SNBUF]
        lsem = rest[SNBUF : 2 * SNBUF]
        asem = rest[2 * SNBUF :]
        cid = lax.axis_index("c")
        sid = lax.axis_index("s")
        w = cid * NS + sid

        base = w * EW
        pltpu.sync_copy(dst_hbm.at[w], dstv)

        zero16 = jnp.zeros((16,), f32)

        def zfill(r, _):
            for u in range(H // 16):
                bufs[0][r, pl.ds(u * 16, 16)] = zero16
            return _

        lax.fori_loop(0, CH, zfill, None)

        nz = N // CH

        def zchunk(kk, _):
            c = sid + kk * NS

            @pl.when(c < nz)
            def _():
                pltpu.sync_copy(bufs[0], macc.at[pl.ds(c * CH, CH)])

            return _

        lax.fori_loop(0, (nz + NS - 1) // NS, zchunk, None)
        plsc.subcore_barrier()

        def wait_l(b):
            pltpu.make_async_copy(h_hbm.at[pl.ds(0, CH)], bufs[b], lsem[b]).wait()

        def wait_a(b):
            pltpu.make_async_copy(bufs[b], macc.at[dstv.at[0]], asem[b]).wait()

        for b in range(SNBUF):
            pltpu.async_copy(h_hbm.at[pl.ds(base + b * CH, CH)], bufs[b], lsem[b])

        def body(k2, _):
            for b in range(SNBUF):
                j = SNBUF * k2 + b

                @pl.when(j < NCHUNK)
                def _scat():
                    wait_l(b)
                    pltpu.async_copy(bufs[b], macc.at[dstv.at[j]], asem[b], add=True)

                @pl.when(j + SNBUF < NCHUNK)
                def _next():
                    wait_a(b)
                    pltpu.async_copy(
                        h_hbm.at[pl.ds(base + (j + SNBUF) * CH, CH)], bufs[b], lsem[b]
                    )

            return _

        lax.fori_loop(0, (NCHUNK + SNBUF - 1) // SNBUF, body, None)
        for b in range(SNBUF):
            wait_a(b)
        plsc.subcore_barrier()

        def wchunk(kk, _):
            c = sid + kk * NS

            @pl.when(c < nz)
            def _():
                pltpu.sync_copy(macc.at[pl.ds(c * CH, CH)], bufs[0])
                pltpu.sync_copy(bufs[0], out_hbm.at[cid, pl.ds(c * CH, CH)])

            return _

        lax.fori_loop(0, (nz + NS - 1) // NS, wchunk, None)

    return k(h, dst3)


def _mm_body(a_ref, w_ref, o_ref):
    o_ref[...] = jnp.dot(a_ref[...], w_ref[...], preferred_element_type=f32)


def _tc_mm(a, wT, tile):
    M = a.shape[0]
    K = a.shape[1]
    return pl.pallas_call(
        _mm_body,
        grid=(M // tile,),
        in_specs=[
            pl.BlockSpec((tile, K), lambda i: (i, 0)),
            pl.BlockSpec((K, H), lambda i: (0, 0)),
        ],
        out_specs=pl.BlockSpec((tile, H), lambda i: (i, 0)),
        out_shape=jax.ShapeDtypeStruct((M, H), f32),
    )(a, wT)


def _edge_init_body(gx_ref, ea_ref, w_ref, o_ref, ob_ref):
    z = jnp.maximum(
        gx_ref[...] + jnp.dot(ea_ref[...], w_ref[...], preferred_element_type=f32),
        0.0,
    )
    o_ref[...] = z
    ob_ref[...] = z.astype(jnp.bfloat16)


def _tc_edge_init(gx, ea, w1eT, tile):
    return pl.pallas_call(
        _edge_init_body,
        grid=(E // tile,),
        in_specs=[
            pl.BlockSpec((tile, H), lambda i: (i, 0)),
            pl.BlockSpec((tile, DE), lambda i: (i, 0)),
            pl.BlockSpec((DE, H), lambda i: (0, 0)),
        ],
        out_specs=[
            pl.BlockSpec((tile, H), lambda i: (i, 0)),
            pl.BlockSpec((tile, H), lambda i: (i, 0)),
        ],
        out_shape=[
            jax.ShapeDtypeStruct((E, H), f32),
            jax.ShapeDtypeStruct((E, H), jnp.bfloat16),
        ],
    )(gx, ea, w1eT)


def _mm_partial_body(p_ref, w_ref, o_ref):
    m = p_ref[0] + p_ref[1]
    o_ref[...] = jnp.dot(m, w_ref[...], preferred_element_type=f32)


def _tc_mm_partials(p, wT, tile):
    return pl.pallas_call(
        _mm_partial_body,
        grid=(N // tile,),
        in_specs=[
            pl.BlockSpec((NC, tile, H), lambda i: (0, i, 0)),
            pl.BlockSpec((H, H), lambda i: (0, 0)),
        ],
        out_specs=pl.BlockSpec((tile, H), lambda i: (i, 0)),
        out_shape=jax.ShapeDtypeStruct((N, H), f32),
    )(p, wT)


def _combine_body(h0_ref, gs_ref, gr_ref, w_ref, o_ref):
    mdirW = gs_ref[...] - jnp.dot(gr_ref[...], w_ref[...], preferred_element_type=f32)
    o_ref[...] = jnp.maximum(h0_ref[...].astype(f32) + mdirW, 0.0)


def _tc_combine(h0, gs, gr, w2T, tile):
    return pl.pallas_call(
        _combine_body,
        grid=(E // tile,),
        in_specs=[
            pl.BlockSpec((tile, H), lambda i: (i, 0)),
            pl.BlockSpec((tile, H), lambda i: (i, 0)),
            pl.BlockSpec((tile, H), lambda i: (i, 0)),
            pl.BlockSpec((H, H), lambda i: (0, 0)),
        ],
        out_specs=pl.BlockSpec((tile, H), lambda i: (i, 0)),
        out_shape=jax.ShapeDtypeStruct((E, H), f32),
    )(h0, gs, gr, w2T)


def _final_body(x_ref, p_ref, b_ref, w3x_ref, w3v_ref, b3_ref, o_ref, sums, counts):
    i = pl.program_id(0)
    nblk = pl.num_programs(0)
    tile = x_ref.shape[0]

    @pl.when(i == 0)
    def _():
        sums[...] = jnp.zeros_like(sums)
        counts[...] = jnp.zeros_like(counts)

    v = p_ref[0] + p_ref[1]
    z = (
        jnp.dot(x_ref[...], w3x_ref[...], preferred_element_type=f32)
        + jnp.dot(v, w3v_ref[...], preferred_element_type=f32)
        + b3_ref[...]
    )
    z = jnp.maximum(z, 0.0)
    onehot = (
        b_ref[...] == lax.broadcasted_iota(jnp.int32, (tile, G), 1)
    ).astype(f32)
    sums[...] += lax.dot_general(
        onehot, z, (((0,), (0,)), ((), ())), preferred_element_type=f32
    )
    counts[...] += lax.dot_general(
        onehot, jnp.ones((tile, H), f32), (((0,), (0,)), ((), ())),
        preferred_element_type=f32,
    )

    @pl.when(i == nblk - 1)
    def _():
        o_ref[...] = sums[...] / jnp.maximum(counts[...], 1.0)


def _tc_final(x, p, batch2, w3xT, w3vT, b3r, tile):
    return pl.pallas_call(
        _final_body,
        grid=(N // tile,),
        in_specs=[
            pl.BlockSpec((tile, DF), lambda i: (i, 0)),
            pl.BlockSpec((NC, tile, H), lambda i: (0, i, 0)),
            pl.BlockSpec((tile, 1), lambda i: (i, 0)),
            pl.BlockSpec((DF, H), lambda i: (0, 0)),
            pl.BlockSpec((H, H), lambda i: (0, 0)),
            pl.BlockSpec((1, H), lambda i: (0, 0)),
        ],
        out_specs=pl.BlockSpec((G, H), lambda i: (0, 0)),
        out_shape=jax.ShapeDtypeStruct((G, H), f32),
        scratch_shapes=[pltpu.VMEM((G, H), f32), pltpu.VMEM((G, H), f32)],
    )(x, p, batch2, w3xT, w3vT, b3r)


def kernel(x, edge_index, revedge_index, edge_attr, num_nodes, batch, W1, W2, W3, b3):
    src = edge_index[0]
    dst = edge_index[1] + (jnp.asarray(num_nodes, jnp.int32) - N)

    src3 = src.reshape(NW, NCHUNK, CH)
    dst3 = dst.reshape(NW, NCHUNK, CH)
    rev3 = revedge_index.reshape(NW, NCHUNK, CH)

    w1xT = W1[:, :DF].T
    w1eT = W1[:, DF:].T
    w2T = W2.T
    w3xT = W3[:, :DF].T
    w3vT = W3[:, DF:].T
    b3r = b3.reshape(1, H)
    batch2 = batch.reshape(N, 1)

    TE = 8000
    TN = 2000

    xW1 = _tc_mm(x, w1xT, TN)
    gx = _sc_gather(xW1, src3)
    h0, h0b = _tc_edge_init(gx, edge_attr, w1eT, TE)

    h = h0
    for _ in range(2):
        p = _sc_segsum(h, dst3)
        mW2 = _tc_mm_partials(p, w2T, TN)
        gs = _sc_gather(mW2, src3)
        gr = _sc_gather(h, rev3)
        h = _tc_combine(h0b, gs, gr, w2T, TE)

    p = _sc_segsum(h, dst3)
    return _tc_final(x, p, batch2, w3xT, w3vT, b3r, 2000)
